# baseline (speedup 1.0000x reference)
import jax
import jax.numpy as jnp
from jax import lax
from jax.experimental import pallas as pl
from jax.experimental.pallas import tpu as pltpu
from jax.experimental.pallas import tpu_sc as plsc

NUM_FEATURES = 26
ROWS_PER_FEATURE = 100000
EMBED_DIM = 16
BATCH = 16384
TBL_ROWS = ROWS_PER_FEATURE * NUM_FEATURES

NUM_CORES = 2
NUM_SUBCORES = 16
NUM_WORKERS = NUM_CORES * NUM_SUBCORES
ROWS_PER_WORKER = BATCH // NUM_WORKERS
CHUNK = 64
NUM_CHUNKS = ROWS_PER_WORKER // CHUNK
LANES = 16

_SUPER = 1024
_SPB = 64
_A_STEPS = 20
_B_START = 19
_B_STEPS = 21
_B_ROW0 = _B_START * _SPB * _SUPER
_F_SPLIT = 13


def _make_sc_body(feats, row_sub, final):
    nf = len(feats)

    def _sc_body(*refs):
        if final:
            (xt_hbm, table_hbm, part_hbm, gamma_hbm, beta_hbm, out_hbm,
             idx_v, buf0, buf1, pbuf, out_v, g_v, b_v, sem0, sem1) = refs
        else:
            (xt_hbm, table_hbm, out_hbm,
             idx_v, buf0, buf1, out_v, sem0, sem1) = refs
        wid = lax.axis_index("s") * NUM_CORES + lax.axis_index("c")
        base = wid * ROWS_PER_WORKER

        if final:
            pltpu.sync_copy(gamma_hbm, g_v)
            pltpu.sync_copy(beta_hbm, b_v)

        for i, f in enumerate(feats):
            pltpu.sync_copy(xt_hbm.at[f, pl.ds(base, ROWS_PER_WORKER)],
                            idx_v.at[i])

        for i, f in enumerate(feats):
            off = jnp.int32(f * ROWS_PER_FEATURE)
            sub = jnp.int32(row_sub)

            def _prep(j, carry, i=i, off=off, sub=sub):
                sl = pl.ds(j * LANES, LANES)
                r = idx_v[i, sl] + off
                r3 = ((r >> 10) << 10) | ((r & 127) << 3) | ((r >> 7) & 7)
                idx_v[i, sl] = r3 - sub
                return carry

            lax.fori_loop(0, ROWS_PER_WORKER // LANES, _prep, None)

        if final:
            g = g_v[...]
            b = b_v[...]
        bufs = (buf0, buf1)
        sems = (sem0, sem1)

        def lane_sum(v):
            return jnp.broadcast_to(jnp.sum(v), (LANES,))

        def fire(c):
            slot = c % 2
            cps = []
            for i in range(nf):
                cps.append(pltpu.async_copy(
                    table_hbm.at[idx_v.at[i, pl.ds(c * CHUNK, CHUNK)]],
                    bufs[slot].at[i], sems[slot]))
            return cps

        pending = {0: fire(0)}
        for c in range(NUM_CHUNKS):
            if c + 1 < NUM_CHUNKS:
                pending[c + 1] = fire(c + 1)
            if final:
                pltpu.sync_copy(part_hbm.at[pl.ds(base + c * CHUNK, CHUNK)],
                                pbuf)
            for cp in pending.pop(c):
                cp.wait()
            buf = bufs[c % 2]

            def row_body(r, carry, buf=buf):
                if final:
                    acc = pbuf[r]
                    lo = 0
                else:
                    acc = buf[0, r]
                    lo = 1
                for i in range(lo, nf):
                    acc = acc + buf[i, r]
                if final:
                    mean = lane_sum(acc) * (1.0 / EMBED_DIM)
                    d = acc - mean
                    yv = lane_sum(d * d) * (1.0 / EMBED_DIM) + 1e-5
                    iv = plsc.bitcast(yv, jnp.int32)
                    iv = jnp.int32(0x5F3759DF) - (iv >> 1)
                    rs = plsc.bitcast(iv, jnp.float32)
                    for _ in range(3):
                        rs = rs * (1.5 - 0.5 * yv * rs * rs)
                    out_v[r] = d * rs * g + b
                else:
                    out_v[r] = acc
                return carry

            lax.fori_loop(0, CHUNK, row_body, None, unroll=4)
            pltpu.sync_copy(out_v, out_hbm.at[pl.ds(base + c * CHUNK, CHUNK)])

    return _sc_body


def _make_sc_call(feats, row_sub, final):
    nf = len(feats)
    scratch = [
        pltpu.VMEM((nf, ROWS_PER_WORKER), jnp.int32),
        pltpu.VMEM((nf, CHUNK, EMBED_DIM), jnp.float32),
        pltpu.VMEM((nf, CHUNK, EMBED_DIM), jnp.float32),
    ]
    if final:
        scratch.append(pltpu.VMEM((CHUNK, EMBED_DIM), jnp.float32))
    scratch += [
        pltpu.VMEM((CHUNK, EMBED_DIM), jnp.float32),
    ]
    if final:
        scratch += [
            pltpu.VMEM((EMBED_DIM,), jnp.float32),
            pltpu.VMEM((EMBED_DIM,), jnp.float32),
        ]
    scratch += [pltpu.SemaphoreType.DMA, pltpu.SemaphoreType.DMA]
    return pl.kernel(
        _make_sc_body(feats, row_sub, final),
        out_type=jax.ShapeDtypeStruct((BATCH, EMBED_DIM), jnp.float32),
        mesh=plsc.VectorSubcoreMesh(core_axis_name="c", subcore_axis_name="s",
                                    num_cores=NUM_CORES,
                                    num_subcores=NUM_SUBCORES),
        compiler_params=pltpu.CompilerParams(needs_layout_passes=False,
                                             use_tc_tiling_on_sc=False),
        scratch_types=scratch,
    )


_sc_call_a = _make_sc_call(tuple(range(_F_SPLIT)), 0, final=False)
_sc_call_b = _make_sc_call(tuple(range(_F_SPLIT, NUM_FEATURES)), _B_ROW0,
                           final=True)


def _tr_body(in_ref, out_ref):
    blk = in_ref[...]
    for g4 in range(_SPB):
        s = jnp.concatenate(
            [blk[:, g4 * _SUPER + g * 128:g4 * _SUPER + (g + 1) * 128]
             for g in range(8)], axis=0)
        out_ref[g4] = s.T


def _make_tr_call(steps, start):
    return pl.pallas_call(
        _tr_body,
        grid=(steps,),
        in_specs=[pl.BlockSpec((EMBED_DIM, _SPB * _SUPER),
                               lambda i, start=start: (0, i + start))],
        out_specs=pl.BlockSpec((_SPB, 128, 128), lambda i: (i, 0, 0)),
        out_shape=jax.ShapeDtypeStruct((steps * _SPB, 128, 128),
                                       jnp.float32),
    )


_tr_a = _make_tr_call(_A_STEPS, 0)
_tr_b = _make_tr_call(_B_STEPS, _B_START)


def kernel(x, table, gamma, beta):
    xt = jnp.transpose(x)
    tt = jnp.transpose(table)
    bt_a = _tr_a(tt)
    bt_b = _tr_b(tt)
    ta = jnp.reshape(bt_a, (_A_STEPS * _SPB * _SUPER, EMBED_DIM))
    tb = jnp.reshape(bt_b, (_B_STEPS * _SPB * _SUPER, EMBED_DIM))
    partial = _sc_call_a(xt, ta)
    return _sc_call_b(xt, tb, partial, gamma, beta)


# speedup vs baseline: 1.1313x; 1.1313x over previous
import jax
import jax.numpy as jnp
from jax import lax
from jax.experimental import pallas as pl
from jax.experimental.pallas import tpu as pltpu
from jax.experimental.pallas import tpu_sc as plsc

NUM_FEATURES = 26
ROWS_PER_FEATURE = 100000
EMBED_DIM = 16
BATCH = 16384

NUM_CORES = 2
NUM_SUBCORES = 16
NUM_WORKERS = NUM_CORES * NUM_SUBCORES
ROWS_PER_WORKER = BATCH // NUM_WORKERS
CHUNK = 128
NUM_CHUNKS = ROWS_PER_WORKER // CHUNK
LANES = 16


def _sc_body(xt_hbm, table_hbm, gamma_hbm, beta_hbm, out_hbm,
             idx_v, buf0, buf1, out_v, g_v, b_v, sem0, sem1):
    wid = lax.axis_index("s") * NUM_CORES + lax.axis_index("c")
    base = wid * ROWS_PER_WORKER

    pltpu.sync_copy(gamma_hbm, g_v)
    pltpu.sync_copy(beta_hbm, b_v)

    for f in range(NUM_FEATURES):
        pltpu.sync_copy(xt_hbm.at[f, pl.ds(base, ROWS_PER_WORKER)],
                        idx_v.at[f])

    for f in range(NUM_FEATURES):
        off = jnp.int32(f * ROWS_PER_FEATURE)

        def _add(j, carry, f=f, off=off):
            sl = pl.ds(j * LANES, LANES)
            r = idx_v[f, sl] + off
            r3 = ((r >> 10) << 10) | ((r & 127) << 3) | ((r >> 7) & 7)
            idx_v[f, sl] = r3
            return carry

        lax.fori_loop(0, ROWS_PER_WORKER // LANES, _add, None)

    g = g_v[...]
    b = b_v[...]
    bufs = (buf0, buf1)
    sems = (sem0, sem1)

    def lane_sum(v):
        return jnp.broadcast_to(jnp.sum(v), (LANES,))

    def fire(c):
        slot = c % 2
        cps = []
        for f in range(NUM_FEATURES):
            cps.append(pltpu.async_copy(
                table_hbm.at[idx_v.at[f, pl.ds(c * CHUNK, CHUNK)]],
                bufs[slot].at[f], sems[slot]))
        return cps

    pending = {0: fire(0)}
    for c in range(NUM_CHUNKS):
        if c + 1 < NUM_CHUNKS:
            pending[c + 1] = fire(c + 1)
        for cp in pending.pop(c):
            cp.wait()
        buf = bufs[c % 2]

        def row_body(r, carry, buf=buf):
            acc = buf[0, r]
            for f in range(1, NUM_FEATURES):
                acc = acc + buf[f, r]
            mean = lane_sum(acc) * (1.0 / EMBED_DIM)
            d = acc - mean
            yv = lane_sum(d * d) * (1.0 / EMBED_DIM) + 1e-5
            iv = plsc.bitcast(yv, jnp.int32)
            iv = jnp.int32(0x5F3759DF) - (iv >> 1)
            rs = plsc.bitcast(iv, jnp.float32)
            for _ in range(3):
                rs = rs * (1.5 - 0.5 * yv * rs * rs)
            out_v[r] = d * rs * g + b
            return carry

        lax.fori_loop(0, CHUNK, row_body, None, unroll=4)
        pltpu.sync_copy(out_v, out_hbm.at[pl.ds(base + c * CHUNK, CHUNK)])


_sc_call = pl.kernel(
    _sc_body,
    out_type=jax.ShapeDtypeStruct((BATCH, EMBED_DIM), jnp.float32),
    mesh=plsc.VectorSubcoreMesh(core_axis_name="c", subcore_axis_name="s",
                                num_cores=NUM_CORES,
                                num_subcores=NUM_SUBCORES),
    compiler_params=pltpu.CompilerParams(needs_layout_passes=False,
                                         use_tc_tiling_on_sc=False),
    scratch_types=[
        pltpu.VMEM((NUM_FEATURES, ROWS_PER_WORKER), jnp.int32),
        pltpu.VMEM((NUM_FEATURES, CHUNK, EMBED_DIM), jnp.float32),
        pltpu.VMEM((NUM_FEATURES, CHUNK, EMBED_DIM), jnp.float32),
        pltpu.VMEM((CHUNK, EMBED_DIM), jnp.float32),
        pltpu.VMEM((EMBED_DIM,), jnp.float32),
        pltpu.VMEM((EMBED_DIM,), jnp.float32),
        pltpu.SemaphoreType.DMA,
        pltpu.SemaphoreType.DMA,
    ],
)


TBL_ROWS = ROWS_PER_FEATURE * NUM_FEATURES
_SUPER = 1024
_SPB = 80
_GRID = (TBL_ROWS + _SPB * _SUPER - 1) // (_SPB * _SUPER)
_NSUPER = _GRID * _SPB


def _tr_body(in_ref, out_ref):
    blk = in_ref[...]
    for g4 in range(_SPB):
        s = jnp.concatenate(
            [blk[:, g4 * _SUPER + g * 128:g4 * _SUPER + (g + 1) * 128]
             for g in range(8)], axis=0)
        out_ref[g4] = s.T


_tr_call = pl.pallas_call(
    _tr_body,
    grid=(_GRID,),
    in_specs=[pl.BlockSpec((EMBED_DIM, _SPB * _SUPER), lambda i: (0, i))],
    out_specs=pl.BlockSpec((_SPB, 128, 128), lambda i: (i, 0, 0)),
    out_shape=jax.ShapeDtypeStruct((_NSUPER, 128, 128), jnp.float32),
)


def kernel(x, table, gamma, beta):
    xt = jnp.transpose(x)
    bt = _tr_call(jnp.transpose(table))
    table_bt = jnp.reshape(bt, (_NSUPER * _SUPER, EMBED_DIM))
    return _sc_call(xt, table_bt, gamma, beta)
